# baseline (speedup 1.0000x reference)
import jax
import jax.numpy as jnp
from jax import lax
from jax.experimental import pallas as pl
from jax.experimental.pallas import tpu as pltpu
from jax.experimental.pallas import tpu_sc as plsc

EMB_DIM = 64
BL = 128
NBUF = 4
LOOK = 2
N_S = 50


def _body(table_hbm, idxt_hbm, o4_hbm, idxc_v, rows, tbufs, gsems, osems,
          nc, bb_per_w):
    wid = lax.axis_index("s") * nc + lax.axis_index("c")
    iota16 = lax.iota(jnp.int32, 16)

    def gather_start(s, j):
        pltpu.async_copy(table_hbm.at[idxc_v.at[s]], rows[j], gsems[j])

    def gather_wait(j):
        pltpu.make_async_copy(table_hbm.at[idxc_v.at[0]], rows[j],
                              gsems[j]).wait()

    def transpose(j):
        for a in range(8):
            for e in range(8):
                d = a * 8 + e
                d_vec = jnp.full((16,), d, jnp.int32)
                for k in range(8):
                    vals = plsc.load_gather(rows[j],
                                            [iota16 + (k * 16), d_vec])
                    tbufs[j][a, e, pl.ds(k * 16, 16)] = vals

    def out_start(s, bb, j):
        pltpu.async_copy(tbufs[j], o4_hbm.at[pl.ds(s * 8, 8), bb], osems[j])

    def out_drain(s, bb, j):
        pltpu.make_async_copy(tbufs[j], o4_hbm.at[pl.ds(s * 8, 8), bb],
                              osems[j]).wait()

    def per_bb(bbi):
        bb = wid * bb_per_w + bbi
        pltpu.sync_copy(idxt_hbm.at[:, pl.ds(bb * BL, BL)], idxc_v)
        for j in range(LOOK):
            gather_start(j, j)

        def outer(t):
            for j in range(NBUF):
                s = t * NBUF + j

                @pl.when(s < N_S)
                def _():
                    gather_wait(j)

                    @pl.when(s >= NBUF)
                    def _():
                        out_drain(s, bb, j)

                    transpose(j)
                    out_start(s, bb, j)
                    s2 = s + LOOK

                    @pl.when(s2 < N_S)
                    def _():
                        gather_start(s2, (j + LOOK) % NBUF)

        pl.loop(0, (N_S + NBUF - 1) // NBUF)(outer)
        for s in range(N_S - NBUF, N_S):
            out_drain(s, bb, s % NBUF)

    pl.loop(0, bb_per_w)(per_bb)


def kernel(inputs, weights):
    n_rows, n_cols = inputs.shape
    assert n_cols == N_S and weights.shape[1] == EMB_DIM

    mesh = plsc.VectorSubcoreMesh(core_axis_name="c", subcore_axis_name="s")
    nw = mesh.num_cores * mesh.num_subcores
    n_bb = n_rows // BL
    bb_per_w = n_bb // nw
    assert bb_per_w * nw * BL == n_rows

    idxt = inputs.astype(jnp.int32).T

    scratch = (
        [pltpu.VMEM((N_S, BL), jnp.int32)]
        + [pltpu.VMEM((BL, EMB_DIM), jnp.float32) for _ in range(NBUF)]
        + [pltpu.VMEM((8, 8, BL), jnp.float32) for _ in range(NBUF)]
        + [pltpu.SemaphoreType.DMA for _ in range(2 * NBUF)]
    )

    def body(table_hbm, idxt_hbm, o4_hbm, idxc_v, *rest):
        rows = rest[:NBUF]
        tbufs = rest[NBUF:2 * NBUF]
        gsems = rest[2 * NBUF:3 * NBUF]
        osems = rest[3 * NBUF:]
        _body(table_hbm, idxt_hbm, o4_hbm, idxc_v, rows, tbufs, gsems,
              osems, mesh.num_cores, bb_per_w)

    n_l = N_S * EMB_DIM // 8
    o4 = pl.kernel(
        body,
        out_type=jax.ShapeDtypeStruct((n_l, n_bb, 8, BL), jnp.float32),
        mesh=mesh,
        scratch_types=scratch,
        compiler_params=pltpu.CompilerParams(use_tc_tiling_on_sc=False,
                                             needs_layout_passes=False),
    )(weights, idxt)

    o6 = o4.reshape(N_S, 8, n_bb, 8, BL).transpose(2, 4, 0, 1, 3)
    return o6.reshape(n_rows, n_cols, EMB_DIM)


# speedup vs baseline: 1.6079x; 1.6079x over previous
import jax
import jax.numpy as jnp
from jax import lax
from jax.experimental import pallas as pl
from jax.experimental.pallas import tpu as pltpu
from jax.experimental.pallas import tpu_sc as plsc

EMB_DIM = 64
BL = 128
NBUF = 4
LOOK = 2
N_S = 50


def _body(table_hbm, idxt_hbm, o4_hbm, idxc_v, rows, tbufs, gsems, osems,
          nc, bb_per_w):
    wid = lax.axis_index("s") * nc + lax.axis_index("c")
    iota16 = lax.iota(jnp.int32, 16)

    def gather_start(s, j):
        pltpu.async_copy(table_hbm.at[idxc_v.at[s]], rows[j], gsems[j])

    def gather_wait(j):
        pltpu.make_async_copy(table_hbm.at[idxc_v.at[0]], rows[j],
                              gsems[j]).wait()

    diag = [(iota16 + jj) & 15 for jj in range(16)]
    diag_a = [dv >> 3 for dv in diag]
    diag_e = [dv & 7 for dv in diag]

    def transpose(j):
        def per_bl0(bl0):
            blv = iota16 + bl0 * 16
            for d0 in range(0, EMB_DIM, 16):
                for jj in range(16):
                    dv = diag[jj] + d0
                    av = diag_a[jj] + (d0 // 8)
                    vals = plsc.load_gather(rows[j], [blv, dv])
                    plsc.store_scatter(tbufs[j], [av, diag_e[jj], blv], vals)
        pl.loop(0, 8)(per_bl0)

    def out_start(s, bb, j):
        pltpu.async_copy(tbufs[j], o4_hbm.at[pl.ds(s * 8, 8), bb], osems[j])

    def out_drain(s, bb, j):
        pltpu.make_async_copy(tbufs[j], o4_hbm.at[pl.ds(s * 8, 8), bb],
                              osems[j]).wait()

    def per_bb(bbi):
        bb = wid * bb_per_w + bbi
        pltpu.sync_copy(idxt_hbm.at[:, pl.ds(bb * BL, BL)], idxc_v)
        for j in range(LOOK):
            gather_start(j, j)

        def outer(t):
            for j in range(NBUF):
                s = t * NBUF + j

                @pl.when(s < N_S)
                def _():
                    gather_wait(j)

                    @pl.when(s >= NBUF)
                    def _():
                        out_drain(s, bb, j)

                    transpose(j)
                    out_start(s, bb, j)
                    s2 = s + LOOK

                    @pl.when(s2 < N_S)
                    def _():
                        gather_start(s2, (j + LOOK) % NBUF)

        pl.loop(0, (N_S + NBUF - 1) // NBUF)(outer)
        for s in range(N_S - NBUF, N_S):
            out_drain(s, bb, s % NBUF)

    pl.loop(0, bb_per_w)(per_bb)


def kernel(inputs, weights):
    n_rows, n_cols = inputs.shape
    assert n_cols == N_S and weights.shape[1] == EMB_DIM

    mesh = plsc.VectorSubcoreMesh(core_axis_name="c", subcore_axis_name="s")
    nw = mesh.num_cores * mesh.num_subcores
    n_bb = n_rows // BL
    bb_per_w = n_bb // nw
    assert bb_per_w * nw * BL == n_rows

    idxt = inputs.astype(jnp.int32).T

    scratch = (
        [pltpu.VMEM((N_S, BL), jnp.int32)]
        + [pltpu.VMEM((BL, EMB_DIM), jnp.float32) for _ in range(NBUF)]
        + [pltpu.VMEM((8, 8, BL), jnp.float32) for _ in range(NBUF)]
        + [pltpu.SemaphoreType.DMA for _ in range(2 * NBUF)]
    )

    def body(table_hbm, idxt_hbm, o4_hbm, idxc_v, *rest):
        rows = rest[:NBUF]
        tbufs = rest[NBUF:2 * NBUF]
        gsems = rest[2 * NBUF:3 * NBUF]
        osems = rest[3 * NBUF:]
        _body(table_hbm, idxt_hbm, o4_hbm, idxc_v, rows, tbufs, gsems,
              osems, mesh.num_cores, bb_per_w)

    n_l = N_S * EMB_DIM // 8
    o4 = pl.kernel(
        body,
        out_type=jax.ShapeDtypeStruct((n_l, n_bb, 8, BL), jnp.float32),
        mesh=mesh,
        scratch_types=scratch,
        compiler_params=pltpu.CompilerParams(use_tc_tiling_on_sc=False,
                                             needs_layout_passes=False),
    )(weights, idxt)

    o6 = o4.reshape(N_S, 8, n_bb, 8, BL).transpose(2, 4, 0, 1, 3)
    return o6.reshape(n_rows, n_cols, EMB_DIM)


# speedup vs baseline: 2.2642x; 1.4082x over previous
import jax
import jax.numpy as jnp
from jax import lax
from jax.experimental import pallas as pl
from jax.experimental.pallas import tpu as pltpu
from jax.experimental.pallas import tpu_sc as plsc

EMB_DIM = 64
BL = 128
NBUF = 4
LOOK = 2
N_S = 50


def _body(table_hbm, idxt_hbm, o4_hbm, idxc_v, rows, tbufs, gsems, osems,
          nc, bb_per_w):
    wid = lax.axis_index("s") * nc + lax.axis_index("c")
    iota16 = lax.iota(jnp.int32, 16)

    def gather_start(s, j):
        pltpu.async_copy(table_hbm.at[idxc_v.at[s]], rows[j], gsems[j])

    def gather_wait(j):
        pltpu.make_async_copy(table_hbm.at[idxc_v.at[0]], rows[j],
                              gsems[j]).wait()

    diag = [(iota16 + jj) & 15 for jj in range(16)]
    diag_a = [dv >> 3 for dv in diag]
    diag_e = [dv & 7 for dv in diag]

    def transpose(j):
        def per_bl0(bl0):
            blv = iota16 + bl0 * 16
            for d0 in range(0, EMB_DIM, 16):
                for jg in (0, 8):
                    vals = [plsc.load_gather(rows[j], [blv, diag[jj] + d0])
                            for jj in range(jg, jg + 8)]
                    for i, jj in enumerate(range(jg, jg + 8)):
                        av = diag_a[jj] + (d0 // 8)
                        plsc.store_scatter(tbufs[j],
                                           [av, diag_e[jj], blv], vals[i])
        pl.loop(0, 8)(per_bl0)

    def out_start(s, bb, j):
        pltpu.async_copy(tbufs[j], o4_hbm.at[pl.ds(s * 8, 8), bb], osems[j])

    def out_drain(s, bb, j):
        pltpu.make_async_copy(tbufs[j], o4_hbm.at[pl.ds(s * 8, 8), bb],
                              osems[j]).wait()

    def per_bb(bbi):
        bb = wid * bb_per_w + bbi
        pltpu.sync_copy(idxt_hbm.at[:, pl.ds(bb * BL, BL)], idxc_v)
        for j in range(LOOK):
            gather_start(j, j)

        def outer(t):
            for j in range(NBUF):
                s = t * NBUF + j

                @pl.when(s < N_S)
                def _():
                    gather_wait(j)

                    @pl.when(s >= NBUF)
                    def _():
                        out_drain(s, bb, j)

                    transpose(j)
                    out_start(s, bb, j)
                    s2 = s + LOOK

                    @pl.when(s2 < N_S)
                    def _():
                        gather_start(s2, (j + LOOK) % NBUF)

        pl.loop(0, (N_S + NBUF - 1) // NBUF)(outer)
        for s in range(N_S - NBUF, N_S):
            out_drain(s, bb, s % NBUF)

    pl.loop(0, bb_per_w)(per_bb)


def kernel(inputs, weights):
    n_rows, n_cols = inputs.shape
    assert n_cols == N_S and weights.shape[1] == EMB_DIM

    mesh = plsc.VectorSubcoreMesh(core_axis_name="c", subcore_axis_name="s")
    nw = mesh.num_cores * mesh.num_subcores
    n_bb = n_rows // BL
    bb_per_w = n_bb // nw
    assert bb_per_w * nw * BL == n_rows

    idxt = inputs.astype(jnp.int32).T

    scratch = (
        [pltpu.VMEM((N_S, BL), jnp.int32)]
        + [pltpu.VMEM((BL, EMB_DIM), jnp.float32) for _ in range(NBUF)]
        + [pltpu.VMEM((8, 8, BL), jnp.float32) for _ in range(NBUF)]
        + [pltpu.SemaphoreType.DMA for _ in range(2 * NBUF)]
    )

    def body(table_hbm, idxt_hbm, o4_hbm, idxc_v, *rest):
        rows = rest[:NBUF]
        tbufs = rest[NBUF:2 * NBUF]
        gsems = rest[2 * NBUF:3 * NBUF]
        osems = rest[3 * NBUF:]
        _body(table_hbm, idxt_hbm, o4_hbm, idxc_v, rows, tbufs, gsems,
              osems, mesh.num_cores, bb_per_w)

    n_l = N_S * EMB_DIM // 8
    o4 = pl.kernel(
        body,
        out_type=jax.ShapeDtypeStruct((n_l, n_bb, 8, BL), jnp.float32),
        mesh=mesh,
        scratch_types=scratch,
        compiler_params=pltpu.CompilerParams(use_tc_tiling_on_sc=False,
                                             needs_layout_passes=False),
    )(weights, idxt)

    o6 = o4.reshape(N_S, 8, n_bb, 8, BL).transpose(2, 4, 0, 1, 3)
    return o6.reshape(n_rows, n_cols, EMB_DIM)


# speedup vs baseline: 3.6471x; 1.6108x over previous
import jax
import jax.numpy as jnp
from jax import lax
from jax.experimental import pallas as pl
from jax.experimental.pallas import tpu as pltpu
from jax.experimental.pallas import tpu_sc as plsc

EMB_DIM = 64
BL = 128
NBUF = 4
LOOK = 2
N_S = 50


def _body(table_hbm, idxt_hbm, o4_hbm, idxc_v, rows, tbufs, gsems, osems,
          nc, bb_per_w):
    wid = lax.axis_index("s") * nc + lax.axis_index("c")
    iota16 = lax.iota(jnp.int32, 16)

    def gather_start(s, j):
        pltpu.async_copy(table_hbm.at[idxc_v.at[s]], rows[j], gsems[j])

    def gather_wait(j):
        pltpu.make_async_copy(table_hbm.at[idxc_v.at[0]], rows[j],
                              gsems[j]).wait()

    diag = [(iota16 + jj) & 15 for jj in range(16)]
    diag_a = [dv >> 3 for dv in diag]
    diag_e = [dv & 7 for dv in diag]

    def transpose(j):
        def per_bl0(bl0):
            blv = iota16 + bl0 * 16
            for d0 in range(0, EMB_DIM, 16):
                for jg in (0, 8):
                    vals = [plsc.load_gather(rows[j], [blv, diag[jj] + d0])
                            for jj in range(jg, jg + 8)]
                    for i, jj in enumerate(range(jg, jg + 8)):
                        av = diag_a[jj] + (d0 // 8)
                        plsc.store_scatter(tbufs[j],
                                           [av, diag_e[jj], blv], vals[i])
        pl.loop(0, 8)(per_bl0)

    def out_start(s, bb, j):
        pltpu.async_copy(tbufs[j], o4_hbm.at[pl.ds(s * 8, 8), bb], osems[j])

    def out_drain(s, bb, j):
        pltpu.make_async_copy(tbufs[j], o4_hbm.at[pl.ds(s * 8, 8), bb],
                              osems[j]).wait()

    def per_bb(bbi):
        bb = wid * bb_per_w + bbi
        pltpu.sync_copy(idxt_hbm.at[:, pl.ds(bb * BL, BL)], idxc_v)
        for j in range(LOOK):
            gather_start(j, j)

        def outer(t):
            for j in range(NBUF):
                s = t * NBUF + j

                @pl.when(s < N_S)
                def _():
                    gather_wait(j)

                    @pl.when(s >= NBUF)
                    def _():
                        out_drain(s, bb, j)

                    transpose(j)
                    out_start(s, bb, j)
                    s2 = s + LOOK

                    @pl.when(s2 < N_S)
                    def _():
                        gather_start(s2, (j + LOOK) % NBUF)

        pl.loop(0, (N_S + NBUF - 1) // NBUF)(outer)
        for s in range(N_S - NBUF, N_S):
            out_drain(s, bb, s % NBUF)

    pl.loop(0, bb_per_w)(per_bb)


def _relayout_body(wt_hbm, wtail_hbm, scr_hbm, tailb, cins, tbs, isems,
                   osems, nc, n_full_blk, max_k):
    del nc
    wid = lax.axis_index("s") * 2 + lax.axis_index("c")
    nw = 32
    iota16 = lax.iota(jnp.int32, 16)
    diag = [(iota16 + jj) & 15 for jj in range(16)]
    e2v = iota16 >> 1
    c2base = (iota16 & 1) * 64
    NB = 3

    def in_start(k, j):
        c0 = (wid + k * nw) * BL
        pltpu.async_copy(wt_hbm.at[:, pl.ds(c0, BL)], cins[j], isems[j])

    def in_wait(j):
        pltpu.make_async_copy(wt_hbm.at[:, pl.ds(0, BL)], cins[j],
                              isems[j]).wait()

    def out_start(k, j):
        rb = wid + k * nw
        pltpu.async_copy(tbs[j], scr_hbm.at[pl.ds(rb * 8, 8)], osems[j])

    def out_drain(j):
        pltpu.make_async_copy(tbs[j], scr_hbm.at[pl.ds(0, 8)],
                              osems[j]).wait()

    def transpose(j):
        def per_rl0(rl0):
            rlv = iota16 + rl0 * 16
            a2v = (iota16 & 0) + rl0
            for d0 in range(0, EMB_DIM, 16):
                for jg in (0, 8):
                    dvs = [diag[jj] + d0 for jj in range(jg, jg + 8)]
                    vals = [plsc.load_gather(cins[j], [dv, rlv])
                            for dv in dvs]
                    for i in range(8):
                        plsc.store_scatter(tbs[j],
                                           [a2v, e2v, c2base + dvs[i]],
                                           vals[i])
        pl.loop(0, 8)(per_rl0)

    for j in range(2):
        @pl.when(wid + j * nw < n_full_blk)
        def _():
            in_start(j, j)

    def outer(t):
        for j in range(NB):
            k = t * NB + j

            @pl.when(wid + k * nw < n_full_blk)
            def _():
                in_wait(j)

                @pl.when(k >= NB)
                def _():
                    out_drain(j)

                transpose(j)
                out_start(k, j)
                k2 = k + 2

                @pl.when(wid + k2 * nw < n_full_blk)
                def _():
                    in_start(k2, (j + 2) % NB)

    pl.loop(0, (max_k + NB - 1) // NB)(outer)
    for k in range(max_k - NB, max_k):
        @pl.when(wid + k * nw < n_full_blk)
        def _():
            out_drain(k % NB)

    @pl.when(wid == 0)
    def _():
        pltpu.sync_copy(wtail_hbm, tailb)
        pltpu.sync_copy(tailb, scr_hbm.at[pl.ds(n_full_blk * 8, 4)])


def kernel(inputs, weights):
    n_rows, n_cols = inputs.shape
    assert n_cols == N_S and weights.shape[1] == EMB_DIM

    mesh = plsc.VectorSubcoreMesh(core_axis_name="c", subcore_axis_name="s")
    nw = mesh.num_cores * mesh.num_subcores
    n_bb = n_rows // BL
    bb_per_w = n_bb // nw
    assert bb_per_w * nw * BL == n_rows

    idxt = inputs.astype(jnp.int32).T

    n_vocab = weights.shape[0]
    n_full_blk = n_vocab // BL
    n_tail = n_vocab - n_full_blk * BL
    max_k = (n_full_blk + nw - 1) // nw

    wt = weights.T
    wtail = weights[n_full_blk * BL:, :].reshape(n_tail * EMB_DIM // 1024,
                                                 8, 128)

    scr1 = (
        [pltpu.VMEM((n_tail * EMB_DIM // 1024, 8, 128), jnp.float32)]
        + [pltpu.VMEM((EMB_DIM, BL), jnp.float32) for _ in range(3)]
        + [pltpu.VMEM((8, 8, BL), jnp.float32) for _ in range(3)]
        + [pltpu.SemaphoreType.DMA for _ in range(6)]
    )

    def body1(wt_hbm, wtail_hbm, scr_hbm, tailb, *rest):
        cins = rest[:3]
        tbs = rest[3:6]
        isems = rest[6:9]
        osems = rest[9:]
        _relayout_body(wt_hbm, wtail_hbm, scr_hbm, tailb, cins, tbs,
                       isems, osems, mesh.num_cores, n_full_blk, max_k)

    scr = pl.kernel(
        body1,
        out_type=jax.ShapeDtypeStruct((n_vocab // 16, 8, 128), jnp.float32),
        mesh=mesh,
        scratch_types=scr1,
        compiler_params=pltpu.CompilerParams(use_tc_tiling_on_sc=True,
                                             needs_layout_passes=False),
    )(wt, wtail)

    weights = scr.reshape(n_vocab, EMB_DIM)

    scratch = (
        [pltpu.VMEM((N_S, BL), jnp.int32)]
        + [pltpu.VMEM((BL, EMB_DIM), jnp.float32) for _ in range(NBUF)]
        + [pltpu.VMEM((8, 8, BL), jnp.float32) for _ in range(NBUF)]
        + [pltpu.SemaphoreType.DMA for _ in range(2 * NBUF)]
    )

    def body(table_hbm, idxt_hbm, o4_hbm, idxc_v, *rest):
        rows = rest[:NBUF]
        tbufs = rest[NBUF:2 * NBUF]
        gsems = rest[2 * NBUF:3 * NBUF]
        osems = rest[3 * NBUF:]
        _body(table_hbm, idxt_hbm, o4_hbm, idxc_v, rows, tbufs, gsems,
              osems, mesh.num_cores, bb_per_w)

    n_l = N_S * EMB_DIM // 8
    o4 = pl.kernel(
        body,
        out_type=jax.ShapeDtypeStruct((n_l, n_bb, 8, BL), jnp.float32),
        mesh=mesh,
        scratch_types=scratch,
        compiler_params=pltpu.CompilerParams(use_tc_tiling_on_sc=False,
                                             needs_layout_passes=False),
    )(weights, idxt)

    o6 = o4.reshape(N_S, 8, n_bb, 8, BL).transpose(2, 4, 0, 1, 3)
    return o6.reshape(n_rows, n_cols, EMB_DIM)


# speedup vs baseline: 3.7827x; 1.0372x over previous
import jax
import jax.numpy as jnp
from jax import lax
from jax.experimental import pallas as pl
from jax.experimental.pallas import tpu as pltpu
from jax.experimental.pallas import tpu_sc as plsc

EMB_DIM = 64
BL = 128
NBUF = 4
LOOK = 2
N_S = 50


def _body(table_hbm, idxt_hbm, o4_hbm, idxc_v, rows, tbufs, gsems, osems,
          nc, bb_per_w):
    wid = lax.axis_index("s") * nc + lax.axis_index("c")
    iota16 = lax.iota(jnp.int32, 16)

    def gather_start(s, j):
        pltpu.async_copy(table_hbm.at[idxc_v.at[s]], rows[j], gsems[j])

    def gather_wait(j):
        pltpu.make_async_copy(table_hbm.at[idxc_v.at[0]], rows[j],
                              gsems[j]).wait()

    diag = [(iota16 + jj) & 15 for jj in range(16)]
    diag_a = [dv >> 3 for dv in diag]
    diag_e = [dv & 7 for dv in diag]

    def transpose(j):
        def per_bl0(bl0):
            blv = iota16 + bl0 * 16
            for d0 in range(0, EMB_DIM, 16):
                tsl = tbufs[j].at[pl.ds(d0 // 8, 2)]
                for jg in (0, 8):
                    vals = [plsc.load_gather(rows[j], [blv, diag[jj] + d0])
                            for jj in range(jg, jg + 8)]
                    for i, jj in enumerate(range(jg, jg + 8)):
                        plsc.store_scatter(tsl,
                                           [diag_a[jj], diag_e[jj], blv],
                                           vals[i])
        pl.loop(0, 8)(per_bl0)

    def out_start(s, bb, j):
        pltpu.async_copy(tbufs[j], o4_hbm.at[pl.ds(s * 8, 8), bb], osems[j])

    def out_drain(s, bb, j):
        pltpu.make_async_copy(tbufs[j], o4_hbm.at[pl.ds(s * 8, 8), bb],
                              osems[j]).wait()

    def per_bb(bbi):
        bb = wid * bb_per_w + bbi
        pltpu.sync_copy(idxt_hbm.at[:, pl.ds(bb * BL, BL)], idxc_v)
        for j in range(LOOK):
            gather_start(j, j)

        def outer(t):
            for j in range(NBUF):
                s = t * NBUF + j

                @pl.when(s < N_S)
                def _():
                    gather_wait(j)

                    @pl.when(s >= NBUF)
                    def _():
                        out_drain(s, bb, j)

                    transpose(j)
                    out_start(s, bb, j)
                    s2 = s + LOOK

                    @pl.when(s2 < N_S)
                    def _():
                        gather_start(s2, (j + LOOK) % NBUF)

        pl.loop(0, (N_S + NBUF - 1) // NBUF)(outer)
        for s in range(N_S - NBUF, N_S):
            out_drain(s, bb, s % NBUF)

    pl.loop(0, bb_per_w)(per_bb)


def _relayout_body(wt_hbm, wtail_hbm, scr_hbm, tailb, cins, tbs, isems,
                   osems, nc, n_full_blk, max_k):
    del nc
    wid = lax.axis_index("s") * 2 + lax.axis_index("c")
    nw = 32
    iota16 = lax.iota(jnp.int32, 16)
    diag = [(iota16 + jj) & 15 for jj in range(16)]
    e2v = iota16 >> 1
    c2base = (iota16 & 1) * 64
    NB = 3

    def in_start(k, j):
        c0 = (wid + k * nw) * BL
        pltpu.async_copy(wt_hbm.at[:, pl.ds(c0, BL)], cins[j], isems[j])

    def in_wait(j):
        pltpu.make_async_copy(wt_hbm.at[:, pl.ds(0, BL)], cins[j],
                              isems[j]).wait()

    def out_start(k, j):
        rb = wid + k * nw
        pltpu.async_copy(tbs[j], scr_hbm.at[pl.ds(rb * 8, 8)], osems[j])

    def out_drain(j):
        pltpu.make_async_copy(tbs[j], scr_hbm.at[pl.ds(0, 8)],
                              osems[j]).wait()

    def transpose(j):
        def per_rl0(rl0):
            rlv = iota16 + rl0 * 16
            a2v = (iota16 & 0) + rl0
            for d0 in range(0, EMB_DIM, 16):
                for jg in (0, 8):
                    dvs = [diag[jj] + d0 for jj in range(jg, jg + 8)]
                    vals = [plsc.load_gather(cins[j], [dv, rlv])
                            for dv in dvs]
                    for i in range(8):
                        plsc.store_scatter(tbs[j],
                                           [a2v, e2v, c2base + dvs[i]],
                                           vals[i])
        pl.loop(0, 8)(per_rl0)

    for j in range(2):
        @pl.when(wid + j * nw < n_full_blk)
        def _():
            in_start(j, j)

    def outer(t):
        for j in range(NB):
            k = t * NB + j

            @pl.when(wid + k * nw < n_full_blk)
            def _():
                in_wait(j)

                @pl.when(k >= NB)
                def _():
                    out_drain(j)

                transpose(j)
                out_start(k, j)
                k2 = k + 2

                @pl.when(wid + k2 * nw < n_full_blk)
                def _():
                    in_start(k2, (j + 2) % NB)

    pl.loop(0, (max_k + NB - 1) // NB)(outer)
    for k in range(max_k - NB, max_k):
        @pl.when(wid + k * nw < n_full_blk)
        def _():
            out_drain(k % NB)

    @pl.when(wid == 0)
    def _():
        pltpu.sync_copy(wtail_hbm, tailb)
        pltpu.sync_copy(tailb, scr_hbm.at[pl.ds(n_full_blk * 8, 4)])


def kernel(inputs, weights):
    n_rows, n_cols = inputs.shape
    assert n_cols == N_S and weights.shape[1] == EMB_DIM

    mesh = plsc.VectorSubcoreMesh(core_axis_name="c", subcore_axis_name="s")
    nw = mesh.num_cores * mesh.num_subcores
    n_bb = n_rows // BL
    bb_per_w = n_bb // nw
    assert bb_per_w * nw * BL == n_rows

    idxt = inputs.astype(jnp.int32).T

    n_vocab = weights.shape[0]
    n_full_blk = n_vocab // BL
    n_tail = n_vocab - n_full_blk * BL
    max_k = (n_full_blk + nw - 1) // nw

    wt = weights.T
    wtail = weights[n_full_blk * BL:, :].reshape(n_tail * EMB_DIM // 1024,
                                                 8, 128)

    scr1 = (
        [pltpu.VMEM((n_tail * EMB_DIM // 1024, 8, 128), jnp.float32)]
        + [pltpu.VMEM((EMB_DIM, BL), jnp.float32) for _ in range(3)]
        + [pltpu.VMEM((8, 8, BL), jnp.float32) for _ in range(3)]
        + [pltpu.SemaphoreType.DMA for _ in range(6)]
    )

    def body1(wt_hbm, wtail_hbm, scr_hbm, tailb, *rest):
        cins = rest[:3]
        tbs = rest[3:6]
        isems = rest[6:9]
        osems = rest[9:]
        _relayout_body(wt_hbm, wtail_hbm, scr_hbm, tailb, cins, tbs,
                       isems, osems, mesh.num_cores, n_full_blk, max_k)

    scr = pl.kernel(
        body1,
        out_type=jax.ShapeDtypeStruct((n_vocab // 16, 8, 128), jnp.float32),
        mesh=mesh,
        scratch_types=scr1,
        compiler_params=pltpu.CompilerParams(use_tc_tiling_on_sc=True,
                                             needs_layout_passes=False),
    )(wt, wtail)

    weights = scr.reshape(n_vocab, EMB_DIM)

    scratch = (
        [pltpu.VMEM((N_S, BL), jnp.int32)]
        + [pltpu.VMEM((BL, EMB_DIM), jnp.float32) for _ in range(NBUF)]
        + [pltpu.VMEM((8, 8, BL), jnp.float32) for _ in range(NBUF)]
        + [pltpu.SemaphoreType.DMA for _ in range(2 * NBUF)]
    )

    def body(table_hbm, idxt_hbm, o4_hbm, idxc_v, *rest):
        rows = rest[:NBUF]
        tbufs = rest[NBUF:2 * NBUF]
        gsems = rest[2 * NBUF:3 * NBUF]
        osems = rest[3 * NBUF:]
        _body(table_hbm, idxt_hbm, o4_hbm, idxc_v, rows, tbufs, gsems,
              osems, mesh.num_cores, bb_per_w)

    n_l = N_S * EMB_DIM // 8
    o4 = pl.kernel(
        body,
        out_type=jax.ShapeDtypeStruct((n_l, n_bb, 8, BL), jnp.float32),
        mesh=mesh,
        scratch_types=scratch,
        compiler_params=pltpu.CompilerParams(use_tc_tiling_on_sc=False,
                                             needs_layout_passes=False),
    )(weights, idxt)

    o6 = o4.reshape(N_S, 8, n_bb, 8, BL).transpose(2, 4, 0, 1, 3)
    return o6.reshape(n_rows, n_cols, EMB_DIM)


# speedup vs baseline: 4.1705x; 1.1025x over previous
import jax
import jax.numpy as jnp
from jax import lax
from jax.experimental import pallas as pl
from jax.experimental.pallas import tpu as pltpu
from jax.experimental.pallas import tpu_sc as plsc

EMB_DIM = 64
BL = 128
NBUF = 4
LOOK = 2
N_S = 50


def _body(table_hbm, idxt_hbm, o4_hbm, idxc_v, rows, tbufs, gsems, osems,
          nc, bb_per_w):
    wid = lax.axis_index("s") * nc + lax.axis_index("c")
    iota16 = lax.iota(jnp.int32, 16)

    def gather_start(s, j):
        pltpu.async_copy(table_hbm.at[idxc_v.at[s]], rows[j], gsems[j])

    def gather_wait(j):
        pltpu.make_async_copy(table_hbm.at[idxc_v.at[0]], rows[j],
                              gsems[j]).wait()

    diag = [(iota16 + jj) & 15 for jj in range(16)]
    diag_a = [dv >> 3 for dv in diag]
    diag_e = [dv & 7 for dv in diag]

    def transpose(j):
        def per_bl0(bl0):
            blv = iota16 + bl0 * 16
            for d0 in range(0, EMB_DIM, 16):
                tsl = tbufs[j].at[pl.ds(d0 // 8, 2)]
                for jg in (0, 8):
                    vals = [plsc.load_gather(rows[j], [blv, diag[jj] + d0])
                            for jj in range(jg, jg + 8)]
                    for i, jj in enumerate(range(jg, jg + 8)):
                        plsc.store_scatter(tsl,
                                           [diag_a[jj], diag_e[jj], blv],
                                           vals[i])
        pl.loop(0, 8)(per_bl0)

    def out_start(s, bb, j):
        pltpu.async_copy(tbufs[j], o4_hbm.at[pl.ds(s * 8, 8), bb], osems[j])

    def out_drain(s, bb, j):
        pltpu.make_async_copy(tbufs[j], o4_hbm.at[pl.ds(s * 8, 8), bb],
                              osems[j]).wait()

    def per_bb(bbi):
        bb = wid * bb_per_w + bbi
        pltpu.sync_copy(idxt_hbm.at[:, pl.ds(bb * BL, BL)], idxc_v)
        for j in range(LOOK):
            gather_start(j, j)

        def outer(t):
            for j in range(NBUF):
                s = t * NBUF + j

                @pl.when(s < N_S)
                def _():
                    gather_wait(j)

                    @pl.when(s >= NBUF)
                    def _():
                        out_drain(s, bb, j)

                    transpose(j)
                    out_start(s, bb, j)
                    s2 = s + LOOK

                    @pl.when(s2 < N_S)
                    def _():
                        gather_start(s2, (j + LOOK) % NBUF)

        pl.loop(0, (N_S + NBUF - 1) // NBUF)(outer)
        for s in range(N_S - NBUF, N_S):
            out_drain(s, bb, s % NBUF)

    pl.loop(0, bb_per_w)(per_bb)


def _relayout_body(wt_hbm, wtail_hbm, scr_hbm, tailb, cins, tbs, isems,
                   osems, nc, n_full_blk, max_k):
    del nc
    wid = lax.axis_index("s") * 2 + lax.axis_index("c")
    nw = 32
    iota16 = lax.iota(jnp.int32, 16)
    diag = [(iota16 + jj) & 15 for jj in range(16)]
    e2v = iota16 >> 1
    c2base = (iota16 & 1) * 64
    NB = 3
    RBL = 256

    def in_start(k, j):
        c0 = (wid + k * nw) * RBL
        pltpu.async_copy(wt_hbm.at[:, pl.ds(c0, RBL)], cins[j], isems[j])

    def in_wait(j):
        pltpu.make_async_copy(wt_hbm.at[:, pl.ds(0, RBL)], cins[j],
                              isems[j]).wait()

    def out_start(k, j):
        rb = wid + k * nw
        pltpu.async_copy(tbs[j], scr_hbm.at[pl.ds(rb * 16, 16)], osems[j])

    def out_drain(j):
        pltpu.make_async_copy(tbs[j], scr_hbm.at[pl.ds(0, 16)],
                              osems[j]).wait()

    def transpose(j):
        def per_rl0(rl0):
            rlv = iota16 + rl0 * 16
            a2v = (iota16 & 0) + rl0
            for d0 in range(0, EMB_DIM, 16):
                for jg in (0, 8):
                    dvs = [diag[jj] + d0 for jj in range(jg, jg + 8)]
                    vals = [plsc.load_gather(cins[j], [dv, rlv])
                            for dv in dvs]
                    for i in range(8):
                        plsc.store_scatter(tbs[j],
                                           [a2v, e2v, c2base + dvs[i]],
                                           vals[i])
        pl.loop(0, RBL // 16)(per_rl0)

    for j in range(2):
        @pl.when(wid + j * nw < n_full_blk)
        def _():
            in_start(j, j)

    def outer(t):
        for j in range(NB):
            k = t * NB + j

            @pl.when(wid + k * nw < n_full_blk)
            def _():
                in_wait(j)

                @pl.when(k >= NB)
                def _():
                    out_drain(j)

                transpose(j)
                out_start(k, j)
                k2 = k + 2

                @pl.when(wid + k2 * nw < n_full_blk)
                def _():
                    in_start(k2, (j + 2) % NB)

    pl.loop(0, (max_k + NB - 1) // NB)(outer)
    for k in range(max_k - NB, max_k):
        @pl.when(wid + k * nw < n_full_blk)
        def _():
            out_drain(k % NB)

    @pl.when(wid == 0)
    def _():
        pltpu.sync_copy(wtail_hbm, tailb)
        pltpu.sync_copy(tailb, scr_hbm.at[pl.ds(n_full_blk * 16, 4)])


def kernel(inputs, weights):
    n_rows, n_cols = inputs.shape
    assert n_cols == N_S and weights.shape[1] == EMB_DIM

    mesh = plsc.VectorSubcoreMesh(core_axis_name="c", subcore_axis_name="s")
    nw = mesh.num_cores * mesh.num_subcores
    n_bb = n_rows // BL
    bb_per_w = n_bb // nw
    assert bb_per_w * nw * BL == n_rows

    idxt = inputs.astype(jnp.int32).T

    n_vocab = weights.shape[0]
    n_full_blk = n_vocab // 256
    n_tail = n_vocab - n_full_blk * 256
    max_k = (n_full_blk + nw - 1) // nw

    wt = weights.T
    wtail = weights[n_full_blk * 256:, :].reshape(n_tail * EMB_DIM // 1024,
                                                  8, 128)

    scr1 = (
        [pltpu.VMEM((n_tail * EMB_DIM // 1024, 8, 128), jnp.float32)]
        + [pltpu.VMEM((EMB_DIM, 256), jnp.float32) for _ in range(3)]
        + [pltpu.VMEM((16, 8, BL), jnp.float32) for _ in range(3)]
        + [pltpu.SemaphoreType.DMA for _ in range(6)]
    )

    def body1(wt_hbm, wtail_hbm, scr_hbm, tailb, *rest):
        cins = rest[:3]
        tbs = rest[3:6]
        isems = rest[6:9]
        osems = rest[9:]
        _relayout_body(wt_hbm, wtail_hbm, scr_hbm, tailb, cins, tbs,
                       isems, osems, mesh.num_cores, n_full_blk, max_k)

    scr = pl.kernel(
        body1,
        out_type=jax.ShapeDtypeStruct((n_vocab // 16, 8, 128), jnp.float32),
        mesh=mesh,
        scratch_types=scr1,
        compiler_params=pltpu.CompilerParams(use_tc_tiling_on_sc=True,
                                             needs_layout_passes=False),
    )(wt, wtail)

    weights = scr.reshape(n_vocab, EMB_DIM)

    scratch = (
        [pltpu.VMEM((N_S, BL), jnp.int32)]
        + [pltpu.VMEM((BL, EMB_DIM), jnp.float32) for _ in range(NBUF)]
        + [pltpu.VMEM((8, 8, BL), jnp.float32) for _ in range(NBUF)]
        + [pltpu.SemaphoreType.DMA for _ in range(2 * NBUF)]
    )

    def body(table_hbm, idxt_hbm, o4_hbm, idxc_v, *rest):
        rows = rest[:NBUF]
        tbufs = rest[NBUF:2 * NBUF]
        gsems = rest[2 * NBUF:3 * NBUF]
        osems = rest[3 * NBUF:]
        _body(table_hbm, idxt_hbm, o4_hbm, idxc_v, rows, tbufs, gsems,
              osems, mesh.num_cores, bb_per_w)

    n_l = N_S * EMB_DIM // 8
    o4 = pl.kernel(
        body,
        out_type=jax.ShapeDtypeStruct((n_l, n_bb, 8, BL), jnp.float32),
        mesh=mesh,
        scratch_types=scratch,
        compiler_params=pltpu.CompilerParams(use_tc_tiling_on_sc=False,
                                             needs_layout_passes=False),
    )(weights, idxt)

    o6 = o4.reshape(N_S, 8, n_bb, 8, BL).transpose(2, 4, 0, 1, 3)
    return o6.reshape(n_rows, n_cols, EMB_DIM)


# speedup vs baseline: 4.2340x; 1.0152x over previous
import jax
import jax.numpy as jnp
from jax import lax
from jax.experimental import pallas as pl
from jax.experimental.pallas import tpu as pltpu
from jax.experimental.pallas import tpu_sc as plsc

EMB_DIM = 64
BL = 128
NBUF = 6
LOOK = 3
N_S = 50


def _body(table_hbm, idxt_hbm, o4_hbm, idxc_v, rows, tbufs, gsems, osems,
          nc, bb_per_w):
    wid = lax.axis_index("s") * nc + lax.axis_index("c")
    iota16 = lax.iota(jnp.int32, 16)

    def gather_start(s, j):
        pltpu.async_copy(table_hbm.at[idxc_v.at[s]], rows[j], gsems[j])

    def gather_wait(j):
        pltpu.make_async_copy(table_hbm.at[idxc_v.at[0]], rows[j],
                              gsems[j]).wait()

    diag = [(iota16 + jj) & 15 for jj in range(16)]
    diag_a = [dv >> 3 for dv in diag]
    diag_e = [dv & 7 for dv in diag]

    def transpose(j):
        def per_bl0(bl0):
            blv = iota16 + bl0 * 16
            for d0 in range(0, EMB_DIM, 16):
                tsl = tbufs[j].at[pl.ds(d0 // 8, 2)]
                for jg in (0, 8):
                    vals = [plsc.load_gather(rows[j], [blv, diag[jj] + d0])
                            for jj in range(jg, jg + 8)]
                    for i, jj in enumerate(range(jg, jg + 8)):
                        plsc.store_scatter(tsl,
                                           [diag_a[jj], diag_e[jj], blv],
                                           vals[i])
        pl.loop(0, 8)(per_bl0)

    def out_start(s, bb, j):
        pltpu.async_copy(tbufs[j], o4_hbm.at[pl.ds(s * 8, 8), bb], osems[j])

    def out_drain(s, bb, j):
        pltpu.make_async_copy(tbufs[j], o4_hbm.at[pl.ds(s * 8, 8), bb],
                              osems[j]).wait()

    def per_bb(bbi):
        bb = wid * bb_per_w + bbi
        pltpu.sync_copy(idxt_hbm.at[:, pl.ds(bb * BL, BL)], idxc_v)
        for j in range(LOOK):
            gather_start(j, j)

        def outer(t):
            for j in range(NBUF):
                s = t * NBUF + j

                @pl.when(s < N_S)
                def _():
                    gather_wait(j)

                    @pl.when(s >= NBUF)
                    def _():
                        out_drain(s, bb, j)

                    transpose(j)
                    out_start(s, bb, j)
                    s2 = s + LOOK

                    @pl.when(s2 < N_S)
                    def _():
                        gather_start(s2, (j + LOOK) % NBUF)

        pl.loop(0, (N_S + NBUF - 1) // NBUF)(outer)
        for s in range(N_S - NBUF, N_S):
            out_drain(s, bb, s % NBUF)

    pl.loop(0, bb_per_w)(per_bb)


def _relayout_body(wt_hbm, wtail_hbm, scr_hbm, tailb, cins, tbs, isems,
                   osems, nc, n_full_blk, max_k):
    del nc
    wid = lax.axis_index("s") * 2 + lax.axis_index("c")
    nw = 32
    iota16 = lax.iota(jnp.int32, 16)
    diag = [(iota16 + jj) & 15 for jj in range(16)]
    e2v = iota16 >> 1
    c2base = (iota16 & 1) * 64
    NB = 3
    RBL = 256

    def in_start(k, j):
        c0 = (wid + k * nw) * RBL
        pltpu.async_copy(wt_hbm.at[:, pl.ds(c0, RBL)], cins[j], isems[j])

    def in_wait(j):
        pltpu.make_async_copy(wt_hbm.at[:, pl.ds(0, RBL)], cins[j],
                              isems[j]).wait()

    def out_start(k, j):
        rb = wid + k * nw
        pltpu.async_copy(tbs[j], scr_hbm.at[pl.ds(rb * 16, 16)], osems[j])

    def out_drain(j):
        pltpu.make_async_copy(tbs[j], scr_hbm.at[pl.ds(0, 16)],
                              osems[j]).wait()

    def transpose(j):
        def per_rl0(rl0):
            rlv = iota16 + rl0 * 16
            a2v = (iota16 & 0) + rl0
            for d0 in range(0, EMB_DIM, 16):
                for jg in (0, 8):
                    dvs = [diag[jj] + d0 for jj in range(jg, jg + 8)]
                    vals = [plsc.load_gather(cins[j], [dv, rlv])
                            for dv in dvs]
                    for i in range(8):
                        plsc.store_scatter(tbs[j],
                                           [a2v, e2v, c2base + dvs[i]],
                                           vals[i])
        pl.loop(0, RBL // 16)(per_rl0)

    for j in range(2):
        @pl.when(wid + j * nw < n_full_blk)
        def _():
            in_start(j, j)

    def outer(t):
        for j in range(NB):
            k = t * NB + j

            @pl.when(wid + k * nw < n_full_blk)
            def _():
                in_wait(j)

                @pl.when(k >= NB)
                def _():
                    out_drain(j)

                transpose(j)
                out_start(k, j)
                k2 = k + 2

                @pl.when(wid + k2 * nw < n_full_blk)
                def _():
                    in_start(k2, (j + 2) % NB)

    pl.loop(0, (max_k + NB - 1) // NB)(outer)
    for k in range(max_k - NB, max_k):
        @pl.when(wid + k * nw < n_full_blk)
        def _():
            out_drain(k % NB)

    @pl.when(wid == 0)
    def _():
        pltpu.sync_copy(wtail_hbm, tailb)
        pltpu.sync_copy(tailb, scr_hbm.at[pl.ds(n_full_blk * 16, 4)])


def kernel(inputs, weights):
    n_rows, n_cols = inputs.shape
    assert n_cols == N_S and weights.shape[1] == EMB_DIM

    mesh = plsc.VectorSubcoreMesh(core_axis_name="c", subcore_axis_name="s")
    nw = mesh.num_cores * mesh.num_subcores
    n_bb = n_rows // BL
    bb_per_w = n_bb // nw
    assert bb_per_w * nw * BL == n_rows

    idxt = inputs.astype(jnp.int32).T

    n_vocab = weights.shape[0]
    n_full_blk = n_vocab // 256
    n_tail = n_vocab - n_full_blk * 256
    max_k = (n_full_blk + nw - 1) // nw

    wt = weights.T
    wtail = weights[n_full_blk * 256:, :].reshape(n_tail * EMB_DIM // 1024,
                                                  8, 128)

    scr1 = (
        [pltpu.VMEM((n_tail * EMB_DIM // 1024, 8, 128), jnp.float32)]
        + [pltpu.VMEM((EMB_DIM, 256), jnp.float32) for _ in range(3)]
        + [pltpu.VMEM((16, 8, BL), jnp.float32) for _ in range(3)]
        + [pltpu.SemaphoreType.DMA for _ in range(6)]
    )

    def body1(wt_hbm, wtail_hbm, scr_hbm, tailb, *rest):
        cins = rest[:3]
        tbs = rest[3:6]
        isems = rest[6:9]
        osems = rest[9:]
        _relayout_body(wt_hbm, wtail_hbm, scr_hbm, tailb, cins, tbs,
                       isems, osems, mesh.num_cores, n_full_blk, max_k)

    scr = pl.kernel(
        body1,
        out_type=jax.ShapeDtypeStruct((n_vocab // 16, 8, 128), jnp.float32),
        mesh=mesh,
        scratch_types=scr1,
        compiler_params=pltpu.CompilerParams(use_tc_tiling_on_sc=True,
                                             needs_layout_passes=False),
    )(wt, wtail)

    weights = scr.reshape(n_vocab, EMB_DIM)

    scratch = (
        [pltpu.VMEM((N_S, BL), jnp.int32)]
        + [pltpu.VMEM((BL, EMB_DIM), jnp.float32) for _ in range(NBUF)]
        + [pltpu.VMEM((8, 8, BL), jnp.float32) for _ in range(NBUF)]
        + [pltpu.SemaphoreType.DMA for _ in range(2 * NBUF)]
    )

    def body(table_hbm, idxt_hbm, o4_hbm, idxc_v, *rest):
        rows = rest[:NBUF]
        tbufs = rest[NBUF:2 * NBUF]
        gsems = rest[2 * NBUF:3 * NBUF]
        osems = rest[3 * NBUF:]
        _body(table_hbm, idxt_hbm, o4_hbm, idxc_v, rows, tbufs, gsems,
              osems, mesh.num_cores, bb_per_w)

    n_l = N_S * EMB_DIM // 8
    o4 = pl.kernel(
        body,
        out_type=jax.ShapeDtypeStruct((n_l, n_bb, 8, BL), jnp.float32),
        mesh=mesh,
        scratch_types=scratch,
        compiler_params=pltpu.CompilerParams(use_tc_tiling_on_sc=False,
                                             needs_layout_passes=False),
    )(weights, idxt)

    o6 = o4.reshape(N_S, 8, n_bb, 8, BL).transpose(2, 4, 0, 1, 3)
    return o6.reshape(n_rows, n_cols, EMB_DIM)
